# baseline (speedup 1.0000x reference)
import functools

import jax
import jax.numpy as jnp
from jax.experimental import pallas as pl
from jax.experimental.pallas import tpu as pltpu
from jax.experimental.pallas import tpu_sc as plsc

NN = 10000
KNN = 16
CHW = 128
NCH = (NN + CHW - 1) // CHW
NP = NCH * CHW
HALF = NN // 2
ROWS = 200
KROWS = 200
MLP_ROWS = 1000
GATHER_W = 128
NEG = -jnp.inf


def _knn_kernel(hT_ref, hr_ref, wr_ref, s_ref, chid_ref,
                xr_ref, sq_ref, *, r0):
    i = pl.program_id(0)

    @pl.when(i == 0)
    def _():
        hT = hT_ref[...]
        sq_ref[...] = jnp.sum(hT * hT, axis=0, keepdims=True)

    h_r = hr_ref[...]
    xr_ref[...] = jnp.dot(h_r, wr_ref[...], preferred_element_type=jnp.float32)

    s = 2.0 * jnp.dot(h_r, hT_ref[...], preferred_element_type=jnp.float32)
    s = s - sq_ref[...]

    col = jax.lax.broadcasted_iota(jnp.int32, (1, NP), 1)
    row_g = (r0 + i * KROWS
             + jax.lax.broadcasted_iota(jnp.int32, (KROWS, 1), 0))
    s = jnp.where((col == row_g) | (col >= NN), NEG, s)
    s_ref[...] = s

    w = jnp.max(s.reshape(KROWS, NCH, CHW), axis=2)
    cio = jax.lax.broadcasted_iota(jnp.int32, (1, NCH), 1)
    ids = []
    for _ in range(KNN):
        m = jnp.max(w, axis=1, keepdims=True)
        c = jnp.min(jnp.where(w == m, cio, NCH), axis=1, keepdims=True)
        ids.append(c)
        w = jnp.where(cio == c, NEG, w)
    chid_ref[...] = jnp.concatenate(ids, axis=1)


def _knn_project(hT, h, wr, r0):
    n, d = h.shape
    do = wr.shape[1]
    grid = HALF // KROWS
    off = r0 // KROWS
    return pl.pallas_call(
        functools.partial(_knn_kernel, r0=r0),
        grid=(grid,),
        in_specs=[
            pl.BlockSpec((d, NP), lambda i: (0, 0)),
            pl.BlockSpec((KROWS, d), lambda i: (i + off, 0)),
            pl.BlockSpec((d, do), lambda i: (0, 0)),
        ],
        out_specs=[
            pl.BlockSpec((KROWS, NP), lambda i: (i, 0)),
            pl.BlockSpec((KROWS, KNN), lambda i: (i, 0)),
            pl.BlockSpec((KROWS, do), lambda i: (i, 0)),
        ],
        out_shape=[
            jax.ShapeDtypeStruct((HALF, NP), jnp.float32),
            jax.ShapeDtypeStruct((HALF, KNN), jnp.int32),
            jax.ShapeDtypeStruct((HALF, do), jnp.float32),
        ],
        scratch_shapes=[pltpu.VMEM((1, NP), jnp.float32)],
    )(hT, h, wr)


def _extract_kernel(cand_ref, chid_ref, nbr_ref):
    lane = jax.lax.broadcasted_iota(jnp.int32, (1, CHW), 1)
    cs, gs = [], []
    for slot in range(KNN):
        cs.append(cand_ref[slot])
        gs.append(chid_ref[:, slot:slot + 1] * CHW + lane)
    c = jnp.concatenate(cs, axis=1)
    g = jnp.concatenate(gs, axis=1)
    outs = []
    for _ in range(KNN):
        m = jnp.max(c, axis=1, keepdims=True)
        idx = jnp.min(jnp.where(c == m, g, NN), axis=1, keepdims=True)
        outs.append(idx)
        c = jnp.where(g == idx, NEG, c)
    nbr_ref[...] = jnp.concatenate(outs, axis=1)


def _extract(cand, chid):
    n = chid.shape[0]
    er = 1000 if n % 1000 == 0 else ROWS
    grid = n // er
    return pl.pallas_call(
        _extract_kernel,
        grid=(grid,),
        in_specs=[
            pl.BlockSpec((KNN, er, CHW), lambda i: (0, i, 0)),
            pl.BlockSpec((er, KNN), lambda i: (i, 0)),
        ],
        out_specs=pl.BlockSpec((er, KNN), lambda i: (i, 0)),
        out_shape=jax.ShapeDtypeStruct((n, KNN), jnp.int32),
    )(cand, chid)


def _sc_gather(table, idx_flat):
    do_full = table.shape[1]
    split = max(1, do_full // 256)
    if split > 1:
        table = table.reshape(table.shape[0] * split, do_full // split)
        idx_flat = (idx_flat[:, None] * split
                    + jnp.arange(split, dtype=idx_flat.dtype)).reshape(-1)
    num_idx = idx_flat.shape[0]
    do = table.shape[1]
    w = GATHER_W
    idx2 = idx_flat.reshape(1, num_idx)
    mesh = plsc.VectorSubcoreMesh(core_axis_name="core",
                                  subcore_axis_name="subcore")

    @pl.kernel(out_type=jax.ShapeDtypeStruct((num_idx, do), table.dtype),
               mesh=mesh)
    def gather_kernel(x_hbm, i_hbm, o_hbm):
        def body(i_vmem, o_vmem):
            pltpu.sync_copy(x_hbm.at[i_vmem.at[0]], o_vmem)

        pltpu.emit_pipeline(
            body,
            grid=(num_idx // w,),
            in_specs=[pl.BlockSpec((1, w), index_map=lambda i: (0, i))],
            out_specs=[pl.BlockSpec((w, do),
                                    index_map=lambda i: (i, 0))],
            core_axis_name="subcore",
            dimension_semantics=(pltpu.PARALLEL,),
        )(i_hbm, o_hbm)

    return gather_kernel(table, idx2)


def _att_kernel(nb_ref, xr_ref, wl_ref, att_ref, b_ref, out_ref):
    xr = xr_ref[...]
    wl = wl_ref[...]
    att = att_ref[...]
    nbs = [jnp.dot(nb_ref[j], wl, preferred_element_type=jnp.float32)
           for j in range(KNN)]
    es = []
    for j in range(KNN):
        t = nbs[j] + xr
        t = jnp.where(t >= 0, t, 0.2 * t)
        es.append(jnp.sum(t * att, axis=1, keepdims=True))
    e = jnp.concatenate(es, axis=1)
    m = jnp.max(e, axis=1, keepdims=True)
    w = jnp.exp(e - m)
    z = jnp.sum(w, axis=1, keepdims=True)
    acc = w[:, 0:1] * nbs[0]
    for j in range(1, KNN):
        acc = acc + w[:, j:j + 1] * nbs[j]
    out_ref[...] = acc / z + b_ref[...]


def _attention(nb, xr, wl_p, att, b):
    n, do = xr.shape
    dp = nb.shape[2]
    grid = n // ROWS
    return pl.pallas_call(
        _att_kernel,
        grid=(grid,),
        in_specs=[
            pl.BlockSpec((KNN, ROWS, dp), lambda i: (0, i, 0)),
            pl.BlockSpec((ROWS, do), lambda i: (i, 0)),
            pl.BlockSpec((dp, do), lambda i: (0, 0)),
            pl.BlockSpec((1, do), lambda i: (0, 0)),
            pl.BlockSpec((1, do), lambda i: (0, 0)),
        ],
        out_specs=pl.BlockSpec((ROWS, do), lambda i: (i, 0)),
        out_shape=jax.ShapeDtypeStruct((n, do), jnp.float32),
    )(nb, xr, wl_p, att.reshape(1, do), b.reshape(1, do))


def _gat_layer(h, wl, wr, att, b):
    n, d = h.shape
    dp = max(128, d)
    h_pad = jnp.pad(h, ((0, 0), (0, dp - d))) if dp != d else h
    wl_p = jnp.pad(wl, ((0, dp - d), (0, 0))) if dp != d else wl
    hT = jnp.pad(h.T, ((0, 0), (0, NP - n)))
    parts = [_knn_project(hT, h, wr, r0) for r0 in (0, HALF)]
    loc = jnp.arange(HALF, dtype=jnp.int32)[:, None] * NCH
    nbrs = []
    for s, chid, _ in parts:
        cflat = (loc + chid).T.reshape(-1)
        cand = _sc_gather(s.reshape(HALF * NCH, CHW), cflat)
        nbrs.append(_extract(cand.reshape(KNN, HALF, CHW), chid))
    outs = []
    for (_, _, xr), nbr in zip(parts, nbrs):
        nb = _sc_gather(h_pad, nbr.T.reshape(-1)).reshape(KNN, HALF, dp)
        outs.append(_attention(nb, xr, wl_p, att, b))
    return jnp.concatenate(outs, axis=0)


def _mlp_kernel(x_ref, w1_ref, b1_ref, w2_ref, b2_ref, w3_ref, b3_ref,
                out_ref):
    z = jnp.dot(x_ref[...], w1_ref[...], preferred_element_type=jnp.float32)
    z = jnp.maximum(z + b1_ref[...], 0.0)
    z = jnp.dot(z, w2_ref[...], preferred_element_type=jnp.float32)
    z = jnp.maximum(z + b2_ref[...], 0.0)
    z = jnp.dot(z, w3_ref[...], preferred_element_type=jnp.float32)
    out_ref[...] = z + b3_ref[...]


def _mlp(x, w1, b1, w2, b2, w3, b3):
    n, d = x.shape
    grid = n // MLP_ROWS
    return pl.pallas_call(
        _mlp_kernel,
        grid=(grid,),
        in_specs=[
            pl.BlockSpec((MLP_ROWS, d), lambda i: (i, 0)),
            pl.BlockSpec(w1.shape, lambda i: (0, 0)),
            pl.BlockSpec((1, w1.shape[1]), lambda i: (0, 0)),
            pl.BlockSpec(w2.shape, lambda i: (0, 0)),
            pl.BlockSpec((1, w2.shape[1]), lambda i: (0, 0)),
            pl.BlockSpec(w3.shape, lambda i: (0, 0)),
            pl.BlockSpec((1, w3.shape[1]), lambda i: (0, 0)),
        ],
        out_specs=pl.BlockSpec((MLP_ROWS, w3.shape[1]), lambda i: (i, 0)),
        out_shape=jax.ShapeDtypeStruct((n, w3.shape[1]), jnp.float32),
    )(x, w1, b1.reshape(1, -1), w2, b2.reshape(1, -1), w3, b3.reshape(1, -1))


def kernel(x, Wl1, Wr1, att1, b1, Wl2, Wr2, att2, b2, Wl3, Wr3, att3, b3,
           Wl4, Wr4, att4, b4, Wm1, bm1, Wm2, bm2, Wm3, bm3):
    h1 = _gat_layer(x, Wl1, Wr1, att1, b1)
    h2 = _gat_layer(h1, Wl2, Wr2, att2, b2)
    h3 = _gat_layer(h2, Wl3, Wr3, att3, b3)
    h4 = _gat_layer(h3, Wl4, Wr4, att4, b4)
    cat = jnp.concatenate([x, h1, h2, h3, h4], axis=1)
    return _mlp(cat, Wm1, bm1, Wm2, bm2, Wm3, bm3)


# speedup vs baseline: 1.0110x; 1.0110x over previous
import functools

import jax
import jax.numpy as jnp
from jax.experimental import pallas as pl
from jax.experimental.pallas import tpu as pltpu
from jax.experimental.pallas import tpu_sc as plsc

NN = 10000
KNN = 16
CHW = 128
NCH = (NN + CHW - 1) // CHW
NP = NCH * CHW
HALF = NN // 2
ROWS = 200
KROWS = 200
MLP_ROWS = 1000
GATHER_W = 128
NEG = -jnp.inf


def _knn_kernel(hT_ref, hr_ref, wr_ref, s_ref, chid_ref,
                xr_ref, sq_ref, *, r0):
    i = pl.program_id(0)

    @pl.when(i == 0)
    def _():
        hT = hT_ref[...]
        sq_ref[...] = jnp.sum(hT * hT, axis=0, keepdims=True)

    h_r = hr_ref[...]
    xr_ref[...] = jnp.dot(h_r, wr_ref[...], preferred_element_type=jnp.float32)

    s = 2.0 * jnp.dot(h_r, hT_ref[...], preferred_element_type=jnp.float32)
    s = s - sq_ref[...]

    col = jax.lax.broadcasted_iota(jnp.int32, (1, NP), 1)
    row_g = (r0 + i * KROWS
             + jax.lax.broadcasted_iota(jnp.int32, (KROWS, 1), 0))
    s = jnp.where((col == row_g) | (col >= NN), NEG, s)
    s_ref[...] = s

    w = jnp.max(s.reshape(KROWS, NCH, CHW), axis=2)
    cio = jax.lax.broadcasted_iota(jnp.int32, (1, NCH), 1)
    ids = []
    for _ in range(KNN):
        m = jnp.max(w, axis=1, keepdims=True)
        c = jnp.min(jnp.where(w == m, cio, NCH), axis=1, keepdims=True)
        ids.append(c)
        w = jnp.where(cio == c, NEG, w)
    chid_ref[...] = jnp.concatenate(ids, axis=1)


def _knn_project(hT, h, wr, r0):
    n, d = h.shape
    do = wr.shape[1]
    grid = HALF // KROWS
    off = r0 // KROWS
    return pl.pallas_call(
        functools.partial(_knn_kernel, r0=r0),
        grid=(grid,),
        in_specs=[
            pl.BlockSpec((d, NP), lambda i: (0, 0)),
            pl.BlockSpec((KROWS, d), lambda i: (i + off, 0)),
            pl.BlockSpec((d, do), lambda i: (0, 0)),
        ],
        out_specs=[
            pl.BlockSpec((KROWS, NP), lambda i: (i, 0)),
            pl.BlockSpec((KROWS, KNN), lambda i: (i, 0)),
            pl.BlockSpec((KROWS, do), lambda i: (i, 0)),
        ],
        out_shape=[
            jax.ShapeDtypeStruct((HALF, NP), jnp.float32),
            jax.ShapeDtypeStruct((HALF, KNN), jnp.int32),
            jax.ShapeDtypeStruct((HALF, do), jnp.float32),
        ],
        scratch_shapes=[pltpu.VMEM((1, NP), jnp.float32)],
    )(hT, h, wr)


def _extract_kernel(cand_ref, chid_ref, nbr_ref):
    lane = jax.lax.broadcasted_iota(jnp.int32, (1, CHW), 1)
    cs, gs = [], []
    for slot in range(KNN):
        cs.append(cand_ref[slot])
        gs.append(chid_ref[:, slot:slot + 1] * CHW + lane)
    c = jnp.concatenate(cs, axis=1)
    g = jnp.concatenate(gs, axis=1)
    outs = []
    for _ in range(KNN):
        m = jnp.max(c, axis=1, keepdims=True)
        idx = jnp.min(jnp.where(c == m, g, NN), axis=1, keepdims=True)
        outs.append(idx)
        c = jnp.where(g == idx, NEG, c)
    nbr_ref[...] = jnp.concatenate(outs, axis=1)


def _extract(cand, chid):
    n = chid.shape[0]
    er = ROWS
    grid = n // er
    return pl.pallas_call(
        _extract_kernel,
        grid=(grid,),
        in_specs=[
            pl.BlockSpec((KNN, er, CHW), lambda i: (0, i, 0)),
            pl.BlockSpec((er, KNN), lambda i: (i, 0)),
        ],
        out_specs=pl.BlockSpec((er, KNN), lambda i: (i, 0)),
        out_shape=jax.ShapeDtypeStruct((n, KNN), jnp.int32),
    )(cand, chid)


def _sc_gather(table, idx_flat):
    do_full = table.shape[1]
    split = max(1, do_full // 256)
    if split > 1:
        table = table.reshape(table.shape[0] * split, do_full // split)
        idx_flat = (idx_flat[:, None] * split
                    + jnp.arange(split, dtype=idx_flat.dtype)).reshape(-1)
    num_idx = idx_flat.shape[0]
    do = table.shape[1]
    w = GATHER_W
    idx2 = idx_flat.reshape(1, num_idx)
    mesh = plsc.VectorSubcoreMesh(core_axis_name="core",
                                  subcore_axis_name="subcore")

    @pl.kernel(out_type=jax.ShapeDtypeStruct((num_idx, do), table.dtype),
               mesh=mesh)
    def gather_kernel(x_hbm, i_hbm, o_hbm):
        def body(i_vmem, o_vmem):
            pltpu.sync_copy(x_hbm.at[i_vmem.at[0]], o_vmem)

        pltpu.emit_pipeline(
            body,
            grid=(num_idx // w,),
            in_specs=[pl.BlockSpec((1, w), index_map=lambda i: (0, i))],
            out_specs=[pl.BlockSpec((w, do),
                                    index_map=lambda i: (i, 0))],
            core_axis_name="subcore",
            dimension_semantics=(pltpu.PARALLEL,),
        )(i_hbm, o_hbm)

    return gather_kernel(table, idx2)


def _att_kernel(nb_ref, xr_ref, wl_ref, att_ref, b_ref, out_ref):
    xr = xr_ref[...]
    wl = wl_ref[...]
    att = att_ref[...]
    nbs = [jnp.dot(nb_ref[j], wl, preferred_element_type=jnp.float32)
           for j in range(KNN)]
    es = []
    for j in range(KNN):
        t = nbs[j] + xr
        t = jnp.where(t >= 0, t, 0.2 * t)
        es.append(jnp.sum(t * att, axis=1, keepdims=True))
    e = jnp.concatenate(es, axis=1)
    m = jnp.max(e, axis=1, keepdims=True)
    w = jnp.exp(e - m)
    z = jnp.sum(w, axis=1, keepdims=True)
    acc = w[:, 0:1] * nbs[0]
    for j in range(1, KNN):
        acc = acc + w[:, j:j + 1] * nbs[j]
    out_ref[...] = acc / z + b_ref[...]


def _attention(nb, xr, wl_p, att, b):
    n, do = xr.shape
    dp = nb.shape[2]
    grid = n // ROWS
    return pl.pallas_call(
        _att_kernel,
        grid=(grid,),
        in_specs=[
            pl.BlockSpec((KNN, ROWS, dp), lambda i: (0, i, 0)),
            pl.BlockSpec((ROWS, do), lambda i: (i, 0)),
            pl.BlockSpec((dp, do), lambda i: (0, 0)),
            pl.BlockSpec((1, do), lambda i: (0, 0)),
            pl.BlockSpec((1, do), lambda i: (0, 0)),
        ],
        out_specs=pl.BlockSpec((ROWS, do), lambda i: (i, 0)),
        out_shape=jax.ShapeDtypeStruct((n, do), jnp.float32),
    )(nb, xr, wl_p, att.reshape(1, do), b.reshape(1, do))


def _gat_layer(h, wl, wr, att, b):
    n, d = h.shape
    dp = max(128, d)
    h_pad = jnp.pad(h, ((0, 0), (0, dp - d))) if dp != d else h
    wl_p = jnp.pad(wl, ((0, dp - d), (0, 0))) if dp != d else wl
    hT = jnp.pad(h.T, ((0, 0), (0, NP - n)))
    parts = [_knn_project(hT, h, wr, r0) for r0 in (0, HALF)]
    loc = jnp.arange(HALF, dtype=jnp.int32)[:, None] * NCH
    nbrs = []
    for s, chid, _ in parts:
        cflat = (loc + chid).T.reshape(-1)
        cand = _sc_gather(s.reshape(HALF * NCH, CHW), cflat)
        nbrs.append(_extract(cand.reshape(KNN, HALF, CHW), chid))
    outs = []
    for (_, _, xr), nbr in zip(parts, nbrs):
        nb = _sc_gather(h_pad, nbr.T.reshape(-1)).reshape(KNN, HALF, dp)
        outs.append(_attention(nb, xr, wl_p, att, b))
    return jnp.concatenate(outs, axis=0)


def _mlp_kernel(x_ref, w1_ref, b1_ref, w2_ref, b2_ref, w3_ref, b3_ref,
                out_ref):
    z = jnp.dot(x_ref[...], w1_ref[...], preferred_element_type=jnp.float32)
    z = jnp.maximum(z + b1_ref[...], 0.0)
    z = jnp.dot(z, w2_ref[...], preferred_element_type=jnp.float32)
    z = jnp.maximum(z + b2_ref[...], 0.0)
    z = jnp.dot(z, w3_ref[...], preferred_element_type=jnp.float32)
    out_ref[...] = z + b3_ref[...]


def _mlp(x, w1, b1, w2, b2, w3, b3):
    n, d = x.shape
    grid = n // MLP_ROWS
    return pl.pallas_call(
        _mlp_kernel,
        grid=(grid,),
        in_specs=[
            pl.BlockSpec((MLP_ROWS, d), lambda i: (i, 0)),
            pl.BlockSpec(w1.shape, lambda i: (0, 0)),
            pl.BlockSpec((1, w1.shape[1]), lambda i: (0, 0)),
            pl.BlockSpec(w2.shape, lambda i: (0, 0)),
            pl.BlockSpec((1, w2.shape[1]), lambda i: (0, 0)),
            pl.BlockSpec(w3.shape, lambda i: (0, 0)),
            pl.BlockSpec((1, w3.shape[1]), lambda i: (0, 0)),
        ],
        out_specs=pl.BlockSpec((MLP_ROWS, w3.shape[1]), lambda i: (i, 0)),
        out_shape=jax.ShapeDtypeStruct((n, w3.shape[1]), jnp.float32),
    )(x, w1, b1.reshape(1, -1), w2, b2.reshape(1, -1), w3, b3.reshape(1, -1))


def kernel(x, Wl1, Wr1, att1, b1, Wl2, Wr2, att2, b2, Wl3, Wr3, att3, b3,
           Wl4, Wr4, att4, b4, Wm1, bm1, Wm2, bm2, Wm3, bm3):
    h1 = _gat_layer(x, Wl1, Wr1, att1, b1)
    h2 = _gat_layer(h1, Wl2, Wr2, att2, b2)
    h3 = _gat_layer(h2, Wl3, Wr3, att3, b3)
    h4 = _gat_layer(h3, Wl4, Wr4, att4, b4)
    cat = jnp.concatenate([x, h1, h2, h3, h4], axis=1)
    return _mlp(cat, Wm1, bm1, Wm2, bm2, Wm3, bm3)
